# baseline (device time: 17861 ns/iter reference)
import jax
import jax.numpy as jnp
from jax import lax
from jax.experimental import pallas as pl
from jax.experimental.pallas import tpu as pltpu

N_CHUNKS = 8


def kernel(x):
    m, n = x.shape
    q = m // N_CHUNKS
    half = N_CHUNKS // 2

    def body(x_hbm, out_hbm, xv, acc, recv_buf, send_sems, recv_sems,
             in_sems, out_sems):
        my_x = lax.axis_index("x")
        my_y = lax.axis_index("y")
        x_nbr = (1 - my_x, my_y)
        y_nbr = (my_x, 1 - my_y)

        def chunk(ref, c):
            return ref.at[pl.ds(c * q, q), :]

        in_copies = {}
        for c in range(N_CHUNKS):
            in_copies[c] = pltpu.make_async_copy(
                chunk(x_hbm, c), chunk(xv, c), in_sems.at[c]
            )
        for c in [c for pair in zip(range(half), range(half, N_CHUNKS))
                  for c in pair]:
            in_copies[c].start()

        barrier_sem = pltpu.get_barrier_semaphore()
        for nbr in (x_nbr, y_nbr):
            pl.semaphore_signal(
                barrier_sem, inc=1,
                device_id=nbr, device_id_type=pl.DeviceIdType.MESH,
            )
        pl.semaphore_wait(barrier_sem, 2)

        def mk_rdma(phase, c, nbr):
            src = chunk(xv if phase == 0 else acc, c)
            sem = N_CHUNKS * phase + c
            return pltpu.make_async_remote_copy(
                src_ref=src,
                dst_ref=recv_buf.at[phase, c],
                send_sem=send_sems.at[sem],
                recv_sem=recv_sems.at[sem],
                device_id=nbr,
                device_id_type=pl.DeviceIdType.MESH,
            )

        nbr0 = [x_nbr] * half + [y_nbr] * half
        nbr1 = [y_nbr] * half + [x_nbr] * half
        order = [c for pair in zip(range(half), range(half, N_CHUNKS))
                 for c in pair]

        p0 = {}
        for c in order:
            in_copies[c].wait()
            p0[c] = mk_rdma(0, c, nbr0[c])
            p0[c].start()
        p1 = {}
        for c in order:
            p0[c].wait()
            chunk(acc, c)[...] = chunk(xv, c)[...] + recv_buf[0, c]
            p1[c] = mk_rdma(1, c, nbr1[c])
            p1[c].start()
        stores = {}
        for c in order:
            p1[c].wait()
            chunk(acc, c)[...] += recv_buf[1, c]
            stores[c] = pltpu.make_async_copy(
                chunk(acc, c), chunk(out_hbm, c), out_sems.at[c]
            )
            stores[c].start()
        for c in order:
            stores[c].wait()

    return pl.pallas_call(
        body,
        out_shape=jax.ShapeDtypeStruct((m, n), jnp.float32),
        in_specs=[pl.BlockSpec(memory_space=pltpu.MemorySpace.HBM)],
        out_specs=pl.BlockSpec(memory_space=pltpu.MemorySpace.HBM),
        scratch_shapes=[
            pltpu.VMEM((m, n), jnp.float32),
            pltpu.VMEM((m, n), jnp.float32),
            pltpu.VMEM((2, N_CHUNKS, q, n), jnp.float32),
            pltpu.SemaphoreType.DMA((2 * N_CHUNKS,)),
            pltpu.SemaphoreType.DMA((2 * N_CHUNKS,)),
            pltpu.SemaphoreType.DMA((N_CHUNKS,)),
            pltpu.SemaphoreType.DMA((N_CHUNKS,)),
        ],
        compiler_params=pltpu.CompilerParams(collective_id=0),
    )(pltpu.with_memory_space_constraint(x, pltpu.MemorySpace.HBM))


# device time: 15299 ns/iter; 1.1675x vs baseline; 1.1675x over previous
import jax
import jax.numpy as jnp
from jax import lax
from jax.experimental import pallas as pl
from jax.experimental.pallas import tpu as pltpu

S = 2
BLK = 128


def kernel(x):
    m, n = x.shape
    h = BLK // S
    HALVES = (0, 1)

    def body(x_hbm, out_hbm, xv, acc, recv1, recv2,
             send_sems, recv_sems, in_sems, out_sems):
        my_x = lax.axis_index("x")
        my_y = lax.axis_index("y")
        x_nbr = (1 - my_x, my_y)
        y_nbr = (my_x, 1 - my_y)

        own_off = {0: my_x * BLK, 1: 2 * BLK + my_y * BLK}
        oth_off = {0: (1 - my_x) * BLK, 1: 2 * BLK + (1 - my_y) * BLK}
        link1 = {0: x_nbr, 1: y_nbr}
        link2 = {0: y_nbr, 1: x_nbr}

        def rows(ref, off, s):
            return ref.at[pl.ds(off + s * h, h), :]

        def sem_idx(k, half, s):
            return (k * 2 + half) * S + s

        ic = {}
        for reg_i, off in ((0, oth_off), (1, own_off)):
            for s in range(S):
                for half in HALVES:
                    c = ic[(reg_i, half, s)] = pltpu.make_async_copy(
                        rows(x_hbm, off[half], s),
                        rows(xv, off[half], s),
                        in_sems.at[(reg_i * 2 + half) * S + s],
                    )
                    c.start()

        barrier_sem = pltpu.get_barrier_semaphore()
        for nbr in (x_nbr, y_nbr):
            pl.semaphore_signal(
                barrier_sem, inc=1,
                device_id=nbr, device_id_type=pl.DeviceIdType.MESH,
            )
        pl.semaphore_wait(barrier_sem, 2)

        k1 = {}
        for s in range(S):
            for half in HALVES:
                ic[(0, half, s)].wait()
                k1[(half, s)] = pltpu.make_async_remote_copy(
                    src_ref=rows(xv, oth_off[half], s),
                    dst_ref=recv1.at[half, s],
                    send_sem=send_sems.at[sem_idx(0, half, s)],
                    recv_sem=recv_sems.at[sem_idx(0, half, s)],
                    device_id=link1[half],
                    device_id_type=pl.DeviceIdType.MESH,
                )
                k1[(half, s)].start()

        k2 = {}
        for s in range(S):
            for half in HALVES:
                k1[(half, s)].wait()
                ic[(1, half, s)].wait()
                rows(acc, own_off[half], s)[...] = (
                    rows(xv, own_off[half], s)[...] + recv1[half, s]
                )
                k2[(half, s)] = pltpu.make_async_remote_copy(
                    src_ref=rows(acc, own_off[half], s),
                    dst_ref=recv2.at[half, s],
                    send_sem=send_sems.at[sem_idx(1, half, s)],
                    recv_sem=recv_sems.at[sem_idx(1, half, s)],
                    device_id=link2[half],
                    device_id_type=pl.DeviceIdType.MESH,
                )
                k2[(half, s)].start()

        k3, st = {}, {}
        for s in range(S):
            for half in HALVES:
                k2[(half, s)].wait()
                rows(acc, own_off[half], s)[...] += recv2[half, s]
                k3[(half, s)] = pltpu.make_async_remote_copy(
                    src_ref=rows(acc, own_off[half], s),
                    dst_ref=rows(out_hbm, own_off[half], s),
                    send_sem=send_sems.at[sem_idx(2, half, s)],
                    recv_sem=recv_sems.at[sem_idx(2, half, s)],
                    device_id=link1[half],
                    device_id_type=pl.DeviceIdType.MESH,
                )
                k3[(half, s)].start()
                st[(half, s)] = pltpu.make_async_copy(
                    rows(acc, own_off[half], s),
                    rows(out_hbm, own_off[half], s),
                    out_sems.at[half * S + s],
                )
                st[(half, s)].start()

        for s in range(S):
            for half in HALVES:
                k3[(half, s)].wait()
                st[(half, s)].wait()

    return pl.pallas_call(
        body,
        out_shape=jax.ShapeDtypeStruct((m, n), jnp.float32),
        in_specs=[pl.BlockSpec(memory_space=pltpu.MemorySpace.HBM)],
        out_specs=pl.BlockSpec(memory_space=pltpu.MemorySpace.HBM),
        scratch_shapes=[
            pltpu.VMEM((m, n), jnp.float32),
            pltpu.VMEM((m, n), jnp.float32),
            pltpu.VMEM((2, S, h, n), jnp.float32),
            pltpu.VMEM((2, S, h, n), jnp.float32),
            pltpu.SemaphoreType.DMA((3 * 2 * S,)),
            pltpu.SemaphoreType.DMA((3 * 2 * S,)),
            pltpu.SemaphoreType.DMA((2 * 2 * S,)),
            pltpu.SemaphoreType.DMA((2 * S,)),
        ],
        compiler_params=pltpu.CompilerParams(collective_id=0),
    )(pltpu.with_memory_space_constraint(x, pltpu.MemorySpace.HBM))


# device time: 14936 ns/iter; 1.1958x vs baseline; 1.0243x over previous
import jax
import jax.numpy as jnp
from jax import lax
from jax.experimental import pallas as pl
from jax.experimental.pallas import tpu as pltpu

S = 4
BLK = 128


def kernel(x):
    m, n = x.shape
    h = BLK // S
    HALVES = (0, 1)

    def body(x_hbm, out_hbm, xv, acc, recv1, recv2,
             send_sems, recv_sems, in_sems, out_sems):
        my_x = lax.axis_index("x")
        my_y = lax.axis_index("y")
        x_nbr = (1 - my_x, my_y)
        y_nbr = (my_x, 1 - my_y)

        own_off = {0: my_x * BLK, 1: 2 * BLK + my_y * BLK}
        oth_off = {0: (1 - my_x) * BLK, 1: 2 * BLK + (1 - my_y) * BLK}
        link1 = {0: x_nbr, 1: y_nbr}
        link2 = {0: y_nbr, 1: x_nbr}

        def rows(ref, off, s):
            return ref.at[pl.ds(off + s * h, h), :]

        def sem_idx(k, half, s):
            return (k * 2 + half) * S + s

        ic = {}
        for reg_i, off in ((0, oth_off), (1, own_off)):
            for s in range(S):
                for half in HALVES:
                    c = ic[(reg_i, half, s)] = pltpu.make_async_copy(
                        rows(x_hbm, off[half], s),
                        rows(xv, off[half], s),
                        in_sems.at[(reg_i * 2 + half) * S + s],
                    )
                    c.start()

        barrier_sem = pltpu.get_barrier_semaphore()
        for nbr in (x_nbr, y_nbr):
            pl.semaphore_signal(
                barrier_sem, inc=1,
                device_id=nbr, device_id_type=pl.DeviceIdType.MESH,
            )
        pl.semaphore_wait(barrier_sem, 2)

        k1 = {}
        for s in range(S):
            for half in HALVES:
                ic[(0, half, s)].wait()
                k1[(half, s)] = pltpu.make_async_remote_copy(
                    src_ref=rows(xv, oth_off[half], s),
                    dst_ref=recv1.at[half, s],
                    send_sem=send_sems.at[sem_idx(0, half, s)],
                    recv_sem=recv_sems.at[sem_idx(0, half, s)],
                    device_id=link1[half],
                    device_id_type=pl.DeviceIdType.MESH,
                )
                k1[(half, s)].start()

        k2 = {}
        for s in range(S):
            for half in HALVES:
                k1[(half, s)].wait()
                ic[(1, half, s)].wait()
                rows(acc, own_off[half], s)[...] = (
                    rows(xv, own_off[half], s)[...] + recv1[half, s]
                )
                k2[(half, s)] = pltpu.make_async_remote_copy(
                    src_ref=rows(acc, own_off[half], s),
                    dst_ref=recv2.at[half, s],
                    send_sem=send_sems.at[sem_idx(1, half, s)],
                    recv_sem=recv_sems.at[sem_idx(1, half, s)],
                    device_id=link2[half],
                    device_id_type=pl.DeviceIdType.MESH,
                )
                k2[(half, s)].start()

        k3, st = {}, {}
        for s in range(S):
            for half in HALVES:
                k2[(half, s)].wait()
                rows(acc, own_off[half], s)[...] += recv2[half, s]
                k3[(half, s)] = pltpu.make_async_remote_copy(
                    src_ref=rows(acc, own_off[half], s),
                    dst_ref=rows(out_hbm, own_off[half], s),
                    send_sem=send_sems.at[sem_idx(2, half, s)],
                    recv_sem=recv_sems.at[sem_idx(2, half, s)],
                    device_id=link1[half],
                    device_id_type=pl.DeviceIdType.MESH,
                )
                k3[(half, s)].start()
                st[(half, s)] = pltpu.make_async_copy(
                    rows(acc, own_off[half], s),
                    rows(out_hbm, own_off[half], s),
                    out_sems.at[half * S + s],
                )
                st[(half, s)].start()

        for s in range(S):
            for half in HALVES:
                k3[(half, s)].wait()
                st[(half, s)].wait()

    return pl.pallas_call(
        body,
        out_shape=jax.ShapeDtypeStruct((m, n), jnp.float32),
        in_specs=[pl.BlockSpec(memory_space=pltpu.MemorySpace.HBM)],
        out_specs=pl.BlockSpec(memory_space=pltpu.MemorySpace.HBM),
        scratch_shapes=[
            pltpu.VMEM((m, n), jnp.float32),
            pltpu.VMEM((m, n), jnp.float32),
            pltpu.VMEM((2, S, h, n), jnp.float32),
            pltpu.VMEM((2, S, h, n), jnp.float32),
            pltpu.SemaphoreType.DMA((3 * 2 * S,)),
            pltpu.SemaphoreType.DMA((3 * 2 * S,)),
            pltpu.SemaphoreType.DMA((2 * 2 * S,)),
            pltpu.SemaphoreType.DMA((2 * S,)),
        ],
        compiler_params=pltpu.CompilerParams(collective_id=0),
    )(pltpu.with_memory_space_constraint(x, pltpu.MemorySpace.HBM))
